# baseline (speedup 1.0000x reference)
import functools

import jax
import jax.numpy as jnp
from jax import lax
from jax.experimental import pallas as pl
from jax.experimental.pallas import tpu as pltpu
from jax.experimental.pallas import tpu_sc as plsc

B = 32
C = 384
H = 56
W = 56
WIN = 8


def _permute(x, idx_i32):
    mesh = plsc.VectorSubcoreMesh(core_axis_name="c", subcore_axis_name="s")
    num_cores = mesh.num_cores

    @functools.partial(
        pl.kernel,
        out_type=jax.ShapeDtypeStruct((B, C, H, W), jnp.float32),
        mesh=mesh,
        scratch_types=[
            pltpu.VMEM((C,), jnp.int32),
            pltpu.SemaphoreType.DMA,
        ],
    )
    def k(in_hbm, idx_hbm, out_hbm, idx_v, sem):
        wid = lax.axis_index("s") * num_cores + lax.axis_index("c")
        pltpu.sync_copy(idx_hbm, idx_v)

        def wait_one():
            pltpu.make_async_copy(
                in_hbm.at[0, 0], out_hbm.at[0, 0], sem
            ).wait()

        def body(i, _):
            v = idx_v[pl.ds(i * 16, 16)]
            for j in range(16):
                c = i * 16 + j
                pltpu.async_copy(in_hbm.at[wid, v[j]], out_hbm.at[wid, c], sem)

            @pl.when(i >= 1)
            def _():
                for _j in range(16):
                    wait_one()

            return 0

        lax.fori_loop(0, C // 16, body, 0)

        def drain(c, _):
            wait_one()
            return 0

        lax.fori_loop(0, 16, drain, 0)

    return k(x, idx_i32)


def kernel(input, indices, indices_inverse):
    idx = indices.astype(jnp.int32)
    return _permute(input, idx)


# speedup vs baseline: 15.3958x; 15.3958x over previous
import functools

import jax
import jax.numpy as jnp
from jax import lax
from jax.experimental import pallas as pl
from jax.experimental.pallas import tpu as pltpu
from jax.experimental.pallas import tpu_sc as plsc

B = 32
C = 384
H = 56
W = 56
CH = 8
NCHUNK = C // CH
NPAIR = NCHUNK // 2


def _permute(x, idx_i32):
    mesh = plsc.VectorSubcoreMesh(core_axis_name="c", subcore_axis_name="s")
    num_cores = mesh.num_cores

    @functools.partial(
        pl.kernel,
        out_type=jax.ShapeDtypeStruct((B, C, H, W), jnp.float32),
        mesh=mesh,
        scratch_types=[
            pltpu.VMEM((C + 16,), jnp.int32),
            pltpu.VMEM((2, CH, H, W), jnp.float32),
            pltpu.SemaphoreType.DMA,
            pltpu.SemaphoreType.DMA,
        ],
    )
    def k(in_hbm, idx_hbm, out_hbm, idx_v, buf, gsem, psem):
        wid = lax.axis_index("s") * num_cores + lax.axis_index("c")
        pltpu.sync_copy(idx_hbm, idx_v.at[pl.ds(0, C)])

        def gather_chunk(i, slot):
            v = idx_v[pl.ds(i * CH, 16)]
            for j in range(CH):
                pltpu.async_copy(
                    in_hbm.at[wid, v[j]], buf.at[slot, j], gsem
                )

        def wait_gather(slot):
            for _j in range(CH):
                pltpu.make_async_copy(
                    in_hbm.at[0, 0], buf.at[slot, 0], gsem
                ).wait()

        def put(i, slot):
            pltpu.async_copy(
                buf.at[slot], out_hbm.at[wid, pl.ds(i * CH, CH)], psem
            )

        def wait_put(slot):
            pltpu.make_async_copy(
                buf.at[slot], out_hbm.at[0, pl.ds(0, CH)], psem
            ).wait()

        gather_chunk(0, 0)
        gather_chunk(1, 1)

        def body(p, _):
            i0 = p * 2
            wait_gather(0)
            put(i0, 0)
            wait_gather(1)
            put(i0 + 1, 1)

            @pl.when(i0 + 2 < NCHUNK)
            def _():
                wait_put(0)
                gather_chunk(i0 + 2, 0)

            @pl.when(i0 + 3 < NCHUNK)
            def _():
                wait_put(1)
                gather_chunk(i0 + 3, 1)

            return 0

        lax.fori_loop(0, NPAIR, body, 0)
        wait_put(0)
        wait_put(1)

    return k(x, idx_i32)


def kernel(input, indices, indices_inverse):
    idx = indices.astype(jnp.int32)
    return _permute(input, idx)


# speedup vs baseline: 15.4154x; 1.0013x over previous
import functools

import jax
import jax.numpy as jnp
from jax import lax
from jax.experimental import pallas as pl
from jax.experimental.pallas import tpu as pltpu
from jax.experimental.pallas import tpu_sc as plsc

B = 32
C = 384
H = 56
W = 56
CH = 4
RING = 4
NCHUNK = C // CH
NGROUP = NCHUNK // RING


def _permute(x, idx_i32):
    mesh = plsc.VectorSubcoreMesh(core_axis_name="c", subcore_axis_name="s")
    num_cores = mesh.num_cores

    @functools.partial(
        pl.kernel,
        out_type=jax.ShapeDtypeStruct((B, C, H, W), jnp.float32),
        mesh=mesh,
        scratch_types=[
            pltpu.VMEM((C + 16,), jnp.int32),
            pltpu.VMEM((RING, CH, H, W), jnp.float32),
            pltpu.SemaphoreType.DMA,
            pltpu.SemaphoreType.DMA,
        ],
    )
    def k(in_hbm, idx_hbm, out_hbm, idx_v, buf, gsem, psem):
        wid = lax.axis_index("s") * num_cores + lax.axis_index("c")
        pltpu.sync_copy(idx_hbm, idx_v.at[pl.ds(0, C)])

        def gather_chunk(i, slot):
            v = idx_v[pl.ds(i * CH, 16)]
            for j in range(CH):
                pltpu.async_copy(
                    in_hbm.at[wid, v[j]], buf.at[slot, j], gsem
                )

        def wait_gather(slot):
            for _j in range(CH):
                pltpu.make_async_copy(
                    in_hbm.at[0, 0], buf.at[slot, 0], gsem
                ).wait()

        def put(i, slot):
            pltpu.async_copy(
                buf.at[slot], out_hbm.at[wid, pl.ds(i * CH, CH)], psem
            )

        def wait_put(slot):
            pltpu.make_async_copy(
                buf.at[slot], out_hbm.at[0, pl.ds(0, CH)], psem
            ).wait()

        for s in range(RING):
            gather_chunk(s, s)

        def body(q, _):
            i0 = q * RING
            for s in range(RING):
                i = i0 + s
                wait_gather(s)
                put(i, s)

                @pl.when(i + RING < NCHUNK)
                def _():
                    wait_put(s)
                    gather_chunk(i + RING, s)

            return 0

        lax.fori_loop(0, NGROUP, body, 0)
        for s in range(RING):
            wait_put(s)

    return k(x, idx_i32)


def kernel(input, indices, indices_inverse):
    idx = indices.astype(jnp.int32)
    return _permute(input, idx)
